# baseline (speedup 1.0000x reference)
import functools

import jax
import jax.numpy as jnp
from jax import lax
from jax.experimental import pallas as pl
from jax.experimental.pallas import tpu as pltpu
from jax.experimental.pallas import tpu_sc as plsc

_B, _L, _D = 4, 2048, 1024
_NC, _NS = 2, 16
_NW = _NC * _NS

_SCB = 1
_TCB = _B - _SCB
_R1 = _TCB * _L
_CH = 16
_LPW = _L // _NW
_NCHUNK = _LPW // _CH

_mesh = plsc.VectorSubcoreMesh(
    core_axis_name="c", subcore_axis_name="s", num_cores=_NC, num_subcores=_NS
)


@functools.partial(
    pl.kernel,
    out_type=jax.ShapeDtypeStruct((_SCB * _L, _D), jnp.float32),
    mesh=_mesh,
    scratch_types=[
        pltpu.VMEM((_LPW, _D), jnp.float32),
        pltpu.VMEM((_CH, _D), jnp.float32),
        pltpu.VMEM((_CH, _D), jnp.float32),
        pltpu.SemaphoreType.DMA,
        pltpu.SemaphoreType.DMA,
        pltpu.SemaphoreType.DMA,
        pltpu.SemaphoreType.DMA,
        pltpu.SemaphoreType.DMA,
    ],
)
def _pos_add_sc(x_hbm, pos_hbm, out_hbm, pos_v, xa, xb,
                pos_sem, in0, in1, out0, out1):
    wid = lax.axis_index("s") * _NC + lax.axis_index("c")
    lbase = wid * _LPW
    bufs = (xa, xb)
    in_sems = (in0, in1)
    out_sems = (out0, out1)

    pos_cp = pltpu.make_async_copy(pos_hbm.at[pl.ds(lbase, _LPW), :], pos_v,
                                   pos_sem)
    pos_cp.start()

    loads = [
        pltpu.make_async_copy(
            x_hbm.at[pl.ds(_R1 + lbase + k * _CH, _CH), :], bufs[k % 2],
            in_sems[k % 2])
        for k in range(_NCHUNK)
    ]
    stores = [
        pltpu.make_async_copy(
            bufs[k % 2], out_hbm.at[pl.ds(lbase + k * _CH, _CH), :],
            out_sems[k % 2])
        for k in range(_NCHUNK)
    ]

    loads[0].start()
    for k in range(_NCHUNK):
        if k + 1 < _NCHUNK:
            if k >= 1:
                stores[k - 1].wait()
            loads[k + 1].start()
        loads[k].wait()
        if k == 0:
            pos_cp.wait()
        x_v = bufs[k % 2]
        prow = k * _CH

        @plsc.parallel_loop(0, _D, step=16, unroll=2)
        def _(i):
            for r in range(_CH):
                plsc.addupdate(x_v.at[r, pl.ds(i, 16)],
                               pos_v[prow + r, pl.ds(i, 16)])

        stores[k].start()
    stores[_NCHUNK - 2].wait()
    stores[_NCHUNK - 1].wait()


def _tc_body(x_ref, p_ref, o_ref):
    o_ref[...] = x_ref[...] + p_ref[...]


_pos_add_tc = pl.pallas_call(
    _tc_body,
    grid=(_TCB,),
    in_specs=[
        pl.BlockSpec((_L, _D), lambda i: (i, 0)),
        pl.BlockSpec((_L, _D), lambda i: (0, 0)),
    ],
    out_specs=pl.BlockSpec((_L, _D), lambda i: (i, 0)),
    out_shape=jax.ShapeDtypeStruct((_B * _L, _D), jnp.float32),
)


def kernel(x, pos_table):
    x2 = x.reshape(_B * _L, _D)
    out_sc = _pos_add_sc(x2, pos_table)
    out_tc = _pos_add_tc(x2, pos_table)
    out = lax.dynamic_update_slice(out_tc, out_sc, (_R1, 0))
    return out.reshape(x.shape)


# speedup vs baseline: 1.0045x; 1.0045x over previous
import functools

import jax
import jax.numpy as jnp
from jax import lax
from jax.experimental import pallas as pl
from jax.experimental.pallas import tpu as pltpu
from jax.experimental.pallas import tpu_sc as plsc

_B, _L, _D = 4, 2048, 1024
_NC, _NS = 2, 16
_NW = _NC * _NS

_SCB = 1
_TCB = _B - _SCB
_R1 = _TCB * _L
_CH = 8
_LPW = _L // _NW
_NCHUNK = _LPW // _CH
_PH = _LPW // 2

_mesh = plsc.VectorSubcoreMesh(
    core_axis_name="c", subcore_axis_name="s", num_cores=_NC, num_subcores=_NS
)


@functools.partial(
    pl.kernel,
    out_type=jax.ShapeDtypeStruct((_SCB * _L, _D), jnp.float32),
    mesh=_mesh,
    scratch_types=[
        pltpu.VMEM((_LPW, _D), jnp.float32),
        pltpu.VMEM((_CH, _D), jnp.float32),
        pltpu.VMEM((_CH, _D), jnp.float32),
        pltpu.SemaphoreType.DMA,
        pltpu.SemaphoreType.DMA,
        pltpu.SemaphoreType.DMA,
        pltpu.SemaphoreType.DMA,
        pltpu.SemaphoreType.DMA,
        pltpu.SemaphoreType.DMA,
    ],
)
def _pos_add_sc(x_hbm, pos_hbm, out_hbm, pos_v, xa, xb,
                pos_sem0, pos_sem1, in0, in1, out0, out1):
    wid = lax.axis_index("s") * _NC + lax.axis_index("c")
    lbase = wid * _LPW
    bufs = (xa, xb)
    in_sems = (in0, in1)
    out_sems = (out0, out1)

    pos_cp0 = pltpu.make_async_copy(
        pos_hbm.at[pl.ds(lbase, _PH), :], pos_v.at[pl.ds(0, _PH), :],
        pos_sem0)
    pos_cp1 = pltpu.make_async_copy(
        pos_hbm.at[pl.ds(lbase + _PH, _PH), :], pos_v.at[pl.ds(_PH, _PH), :],
        pos_sem1)
    pos_cp0.start()

    loads = [
        pltpu.make_async_copy(
            x_hbm.at[pl.ds(_R1 + lbase + k * _CH, _CH), :], bufs[k % 2],
            in_sems[k % 2])
        for k in range(_NCHUNK)
    ]
    stores = [
        pltpu.make_async_copy(
            bufs[k % 2], out_hbm.at[pl.ds(lbase + k * _CH, _CH), :],
            out_sems[k % 2])
        for k in range(_NCHUNK)
    ]

    loads[0].start()
    pos_cp1.start()
    for k in range(_NCHUNK):
        if k + 1 < _NCHUNK:
            if k >= 1:
                stores[k - 1].wait()
            loads[k + 1].start()
        loads[k].wait()
        if k == 0:
            pos_cp0.wait()
        if k == _PH // _CH:
            pos_cp1.wait()
        x_v = bufs[k % 2]
        prow = k * _CH

        @plsc.parallel_loop(0, _D, step=16, unroll=2)
        def _(i):
            for r in range(_CH):
                plsc.addupdate(x_v.at[r, pl.ds(i, 16)],
                               pos_v[prow + r, pl.ds(i, 16)])

        stores[k].start()
    stores[_NCHUNK - 2].wait()
    stores[_NCHUNK - 1].wait()


def _tc_body(x_ref, p_ref, o_ref):
    o_ref[...] = x_ref[...] + p_ref[...]


_pos_add_tc = pl.pallas_call(
    _tc_body,
    grid=(_TCB,),
    in_specs=[
        pl.BlockSpec((_L, _D), lambda i: (i, 0)),
        pl.BlockSpec((_L, _D), lambda i: (0, 0)),
    ],
    out_specs=pl.BlockSpec((_L, _D), lambda i: (i, 0)),
    out_shape=jax.ShapeDtypeStruct((_B * _L, _D), jnp.float32),
)


def kernel(x, pos_table):
    x2 = x.reshape(_B * _L, _D)
    out_sc = _pos_add_sc(x2, pos_table)
    out_tc = _pos_add_tc(x2, pos_table)
    out = lax.dynamic_update_slice(out_tc, out_sc, (_R1, 0))
    return out.reshape(x.shape)


# speedup vs baseline: 1.1394x; 1.1343x over previous
import functools

import jax
import jax.numpy as jnp
from jax import lax
from jax.experimental import pallas as pl
from jax.experimental.pallas import tpu as pltpu
from jax.experimental.pallas import tpu_sc as plsc

_B, _L, _D = 4, 2048, 1024
_NC, _NS = 2, 16
_NW = _NC * _NS

_SC_ROWS = 1024
_R1 = _B * _L - _SC_ROWS
_PBASE = _L - _SC_ROWS
_CH = 8
_RPW = _SC_ROWS // _NW
_NCHUNK = _RPW // _CH

_mesh = plsc.VectorSubcoreMesh(
    core_axis_name="c", subcore_axis_name="s", num_cores=_NC, num_subcores=_NS
)


@functools.partial(
    pl.kernel,
    out_type=jax.ShapeDtypeStruct((_SC_ROWS, _D), jnp.float32),
    mesh=_mesh,
    scratch_types=[
        pltpu.VMEM((_RPW, _D), jnp.float32),
        pltpu.VMEM((_CH, _D), jnp.float32),
        pltpu.VMEM((_CH, _D), jnp.float32),
        pltpu.SemaphoreType.DMA,
        pltpu.SemaphoreType.DMA,
        pltpu.SemaphoreType.DMA,
        pltpu.SemaphoreType.DMA,
        pltpu.SemaphoreType.DMA,
    ],
)
def _pos_add_sc(x_hbm, pos_hbm, out_hbm, pos_v, xa, xb,
                pos_sem, in0, in1, out0, out1):
    wid = lax.axis_index("s") * _NC + lax.axis_index("c")
    rbase = wid * _RPW
    bufs = (xa, xb)
    in_sems = (in0, in1)
    out_sems = (out0, out1)

    pos_cp = pltpu.make_async_copy(
        pos_hbm.at[pl.ds(_PBASE + rbase, _RPW), :], pos_v, pos_sem)
    pos_cp.start()

    loads = [
        pltpu.make_async_copy(
            x_hbm.at[pl.ds(_R1 + rbase + k * _CH, _CH), :], bufs[k % 2],
            in_sems[k % 2])
        for k in range(_NCHUNK)
    ]
    stores = [
        pltpu.make_async_copy(
            bufs[k % 2], out_hbm.at[pl.ds(rbase + k * _CH, _CH), :],
            out_sems[k % 2])
        for k in range(_NCHUNK)
    ]

    loads[0].start()
    for k in range(_NCHUNK):
        if k + 1 < _NCHUNK:
            if k >= 1:
                stores[k - 1].wait()
            loads[k + 1].start()
        loads[k].wait()
        if k == 0:
            pos_cp.wait()
        x_v = bufs[k % 2]
        prow = k * _CH

        @plsc.parallel_loop(0, _D, step=16, unroll=2)
        def _(i):
            for r in range(_CH):
                plsc.addupdate(x_v.at[r, pl.ds(i, 16)],
                               pos_v[prow + r, pl.ds(i, 16)])

        stores[k].start()
    stores[_NCHUNK - 2].wait()
    stores[_NCHUNK - 1].wait()


_TBLK = 1024


def _tc_body(x_ref, p_hbm, o_ref, p_v, p_sem):
    i = pl.program_id(0)

    @pl.when(i == 0)
    def _():
        cp = pltpu.make_async_copy(p_hbm.at[pl.ds(0, _L), :], p_v, p_sem)
        cp.start()
        cp.wait()

    half = (i % (_L // _TBLK)) * _TBLK
    o_ref[...] = x_ref[...] + p_v[pl.ds(half, _TBLK), :]


_pos_add_tc = pl.pallas_call(
    _tc_body,
    grid=(_R1 // _TBLK,),
    in_specs=[
        pl.BlockSpec((_TBLK, _D), lambda i: (i, 0)),
        pl.BlockSpec(memory_space=pl.ANY),
    ],
    out_specs=pl.BlockSpec((_TBLK, _D), lambda i: (i, 0)),
    out_shape=jax.ShapeDtypeStruct((_B * _L, _D), jnp.float32),
    scratch_shapes=[
        pltpu.VMEM((_L, _D), jnp.float32),
        pltpu.SemaphoreType.DMA,
    ],
)


def kernel(x, pos_table):
    x2 = x.reshape(_B * _L, _D)
    out_sc = _pos_add_sc(x2, pos_table)
    out_tc = _pos_add_tc(x2, pos_table)
    out = lax.dynamic_update_slice(out_tc, out_sc, (_R1, 0))
    return out.reshape(x.shape)
